# baseline (speedup 1.0000x reference)
import jax
import jax.numpy as jnp
from jax import lax
from jax.experimental import pallas as pl
from jax.experimental.pallas import tpu as pltpu
from jax.experimental.pallas import tpu_sc as plsc

_K = 512
_NCOLS = 2048
_ROWS_PER_BLOCK = 256
_N_WORKERS = 32
_SC_ROWS = 2304


def _swar_count(pg, cand_pair):
    rows = pg.shape[0]
    t = pg - cand_pair
    g = jax.lax.shift_right_logical(t, 15) & jnp.int32(0x00010001)
    gg = g[:, :128]
    for j in range(1, 8):
        gg = gg + g[:, j * 128:(j + 1) * 128]
    both = (gg >> 16) + (gg & jnp.int32(0xFFFF))
    return jnp.sum(both, axis=1, keepdims=True, dtype=jnp.float32)


def _tc_topk_kernel(x_ref, o_ref):
    x = x_ref[...]
    rows = x.shape[0]
    half = _NCOLS // 2
    keys = jax.lax.bitcast_convert_type(x, jnp.int32) & jnp.int32(0x7FFFFFFF)
    kf = jnp.float32(_K)

    hi = keys >> 16
    pg = (hi[:, :half] << 16) | hi[:, half:] | jnp.int32(-0x7FFF8000)
    pre_a = jnp.zeros((rows, 1), jnp.int32)
    for b in range(14, -1, -1):
        cand = pre_a | jnp.int32(1 << b)
        cnt = _swar_count(pg, cand | (cand << 16))
        pre_a = jnp.where(cnt >= kf, cand, pre_a)

    c_hi = jnp.sum(hi > pre_a, axis=1, keepdims=True, dtype=jnp.float32)
    lowm = jax.lax.shift_right_logical(keys, 1) & jnp.int32(0x7FFF)
    ml = jnp.where(hi == pre_a, lowm, 0)
    pg = (ml[:, :half] << 16) | ml[:, half:] | jnp.int32(-0x7FFF8000)
    pre_b = jnp.zeros((rows, 1), jnp.int32)
    for b in range(14, -1, -1):
        cand = pre_b | jnp.int32(1 << b)
        cnt = c_hi + _swar_count(pg, cand | (cand << 16))
        pre_b = jnp.where(cnt >= kf, cand, pre_b)

    t1 = (pre_a << 16) | (pre_b << 1) | jnp.int32(1)
    cnt = jnp.sum(keys >= t1, axis=1, keepdims=True, dtype=jnp.float32)
    thresh = jnp.where(cnt >= kf, t1, t1 - 1)

    cnt = jnp.sum((keys >= thresh).astype(jnp.int32), axis=1, keepdims=True)
    scale = jnp.float32(_NCOLS) / cnt.astype(jnp.float32)
    o_ref[...] = jnp.where(keys >= thresh, x * scale, 0.0)


def _tc_call(flat, row_start):
    n_rows = flat.shape[0]
    nblk = (n_rows - row_start) // _ROWS_PER_BLOCK
    blk0 = row_start // _ROWS_PER_BLOCK
    return pl.pallas_call(
        _tc_topk_kernel,
        grid=(nblk,),
        in_specs=[pl.BlockSpec((_ROWS_PER_BLOCK, _NCOLS),
                               lambda i: (i + blk0, 0))],
        out_specs=pl.BlockSpec((_ROWS_PER_BLOCK, _NCOLS),
                               lambda i: (i + blk0, 0)),
        out_shape=jax.ShapeDtypeStruct((n_rows, _NCOLS), flat.dtype),
    )(flat)


_ABS_MASK = 0x7FFFFFFF


def _sc_scan_level(hist_ref, nb, need):
    nchunks = nb // 16
    cum = jnp.int32(0)
    done = jnp.bool_(False)
    chosen_c = jnp.int32(0)
    base_above = jnp.int32(0)
    for c in range(nchunks - 1, -1, -1):
        sv = hist_ref[pl.ds(c * 16, 16)]
        s = jnp.sum(sv)
        hit = jnp.logical_and(jnp.logical_not(done), cum + s >= need)
        chosen_c = jnp.where(hit, jnp.int32(c), chosen_c)
        base_above = jnp.where(hit, cum, base_above)
        done = jnp.logical_or(done, hit)
        cum = cum + s
    need_local = need - base_above
    sv = hist_ref[pl.ds(chosen_c * 16, 16)]
    rchunk = lax.rev(sv, dimensions=(0,))
    cs = plsc.cumsum(rchunk)
    first = jnp.logical_and(cs >= need_local, cs - rchunk < need_local)
    zeros = jnp.zeros((16,), jnp.int32)
    in_bucket = jnp.sum(jnp.where(first, rchunk, zeros))
    above_in_chunk = jnp.sum(jnp.where(first, cs - rchunk, zeros))
    jrev = jnp.sum(jnp.where(first, lax.iota(jnp.int32, 16), zeros))
    bucket = chosen_c * 16 + (jnp.int32(15) - jrev)
    return bucket, base_above + above_in_chunk, in_bucket


_CHUNK = 8


def _sc_body(x_hbm, o_hbm, row_v, out_v, hist_ref):
    nc = 2
    wid = lax.axis_index("s") * nc + lax.axis_index("c")
    rows_per_w = _SC_ROWS // _N_WORKERS
    ones16 = jnp.ones((16,), jnp.int32)

    def do_row(rr, carry):
        levels = ((23, 256, None), (15, 256, 23), (7, 256, 15), (0, 128, 7))
        pfx = jnp.int32(0)
        need = jnp.int32(_K)
        above_total = jnp.int32(0)
        in_bucket = jnp.int32(0)
        for shift, nb, cmp_shift in levels:
            for j in range(nb // 16):
                hist_ref[pl.ds(j * 16, 16)] = jnp.zeros((16,), jnp.int32)

            def hist_step(i, c, shift=shift, nb=nb, cmp_shift=cmp_shift,
                          pfx=pfx):
                v = row_v[rr, pl.ds(i * 16, 16)]
                k = jax.lax.bitcast_convert_type(v, jnp.int32) & jnp.int32(
                    _ABS_MASK)
                d = (k >> shift) & jnp.int32(nb - 1)
                if cmp_shift is None:
                    m = k >= jnp.int32(0)
                else:
                    m = (k >> cmp_shift) == pfx
                plsc.addupdate_scatter(hist_ref, [d], ones16, mask=m)
                return c

            lax.fori_loop(0, _NCOLS // 16, hist_step, jnp.int32(0), unroll=8)
            bucket, above, inb = _sc_scan_level(hist_ref, nb, need)
            pfx = (pfx << (8 if nb == 256 else 7)) | bucket
            need = need - above
            above_total = above_total + above
            in_bucket = inb

        thresh = pfx
        total = above_total + in_bucket
        total_v = jnp.zeros((16,), jnp.int32) + total
        scale = jnp.full((16,), float(_NCOLS), jnp.float32) / total_v.astype(
            jnp.float32)

        def out_step(i, c):
            v = row_v[rr, pl.ds(i * 16, 16)]
            k = jax.lax.bitcast_convert_type(v, jnp.int32) & jnp.int32(
                _ABS_MASK)
            out_v[rr, pl.ds(i * 16, 16)] = jnp.where(
                k >= thresh, v * scale, 0.0)
            return c

        lax.fori_loop(0, _NCOLS // 16, out_step, jnp.int32(0), unroll=8)
        return carry

    def do_chunk(g, carry):
        base = wid * rows_per_w + g * _CHUNK
        pltpu.sync_copy(x_hbm.at[pl.ds(base, _CHUNK)], row_v)
        lax.fori_loop(0, _CHUNK, do_row, jnp.int32(0))
        pltpu.sync_copy(out_v, o_hbm.at[pl.ds(base, _CHUNK)])
        return carry

    lax.fori_loop(0, rows_per_w // _CHUNK, do_chunk, jnp.int32(0))


def _sc_call(flat):
    mesh = plsc.VectorSubcoreMesh(core_axis_name="c", subcore_axis_name="s")
    f = pl.kernel(
        _sc_body,
        out_type=jax.ShapeDtypeStruct((_SC_ROWS, _NCOLS), jnp.float32),
        mesh=mesh,
        scratch_types=[
            pltpu.VMEM((_CHUNK, _NCOLS), jnp.float32),
            pltpu.VMEM((_CHUNK, _NCOLS), jnp.float32),
            pltpu.VMEM((256,), jnp.int32),
        ],
        compiler_params=pltpu.CompilerParams(needs_layout_passes=False),
    )
    return f(flat)


def kernel(x):
    shape = x.shape
    flat = x.reshape(-1, shape[-1])
    n_rows = flat.shape[0]
    if _SC_ROWS >= n_rows:
        out = _sc_call(flat)
    elif _SC_ROWS == 0:
        out = _tc_call(flat, 0)
    else:
        sc_part = _sc_call(flat)
        out = _tc_call(flat, _SC_ROWS)
        out = lax.dynamic_update_slice(out, sc_part, (0, 0))
    return out.reshape(shape), 0, 0


# speedup vs baseline: 1.0710x; 1.0710x over previous
import jax
import jax.numpy as jnp
from jax import lax
from jax.experimental import pallas as pl
from jax.experimental.pallas import tpu as pltpu
from jax.experimental.pallas import tpu_sc as plsc

_K = 512
_NCOLS = 2048
_ROWS_PER_BLOCK = 512
_N_WORKERS = 32
_SC_ROWS = 2048


def _swar_count(pg, cand_pair):
    rows = pg.shape[0]
    t = pg - cand_pair
    g = jax.lax.shift_right_logical(t, 15) & jnp.int32(0x00010001)
    gg = g[:, :128]
    for j in range(1, 8):
        gg = gg + g[:, j * 128:(j + 1) * 128]
    both = (gg >> 16) + (gg & jnp.int32(0xFFFF))
    return jnp.sum(both, axis=1, keepdims=True, dtype=jnp.float32)


def _tc_topk_kernel(x_ref, o_ref):
    x = x_ref[...]
    rows = x.shape[0]
    half = _NCOLS // 2
    keys = jax.lax.bitcast_convert_type(x, jnp.int32) & jnp.int32(0x7FFFFFFF)
    kf = jnp.float32(_K)

    hi = keys >> 16
    pg = (hi[:, :half] << 16) | hi[:, half:] | jnp.int32(-0x7FFF8000)
    pre_a = jnp.zeros((rows, 1), jnp.int32)
    for b in range(14, -1, -1):
        cand = pre_a | jnp.int32(1 << b)
        cnt = _swar_count(pg, cand | (cand << 16))
        pre_a = jnp.where(cnt >= kf, cand, pre_a)

    c_hi = jnp.sum(hi > pre_a, axis=1, keepdims=True, dtype=jnp.float32)
    lowm = jax.lax.shift_right_logical(keys, 1) & jnp.int32(0x7FFF)
    ml = jnp.where(hi == pre_a, lowm, 0)
    pg = (ml[:, :half] << 16) | ml[:, half:] | jnp.int32(-0x7FFF8000)
    pre_b = jnp.zeros((rows, 1), jnp.int32)
    for b in range(14, -1, -1):
        cand = pre_b | jnp.int32(1 << b)
        cnt = c_hi + _swar_count(pg, cand | (cand << 16))
        pre_b = jnp.where(cnt >= kf, cand, pre_b)

    t1 = (pre_a << 16) | (pre_b << 1) | jnp.int32(1)
    cnt = jnp.sum(keys >= t1, axis=1, keepdims=True, dtype=jnp.float32)
    thresh = jnp.where(cnt >= kf, t1, t1 - 1)

    cnt = jnp.sum((keys >= thresh).astype(jnp.int32), axis=1, keepdims=True)
    scale = jnp.float32(_NCOLS) / cnt.astype(jnp.float32)
    o_ref[...] = jnp.where(keys >= thresh, x * scale, 0.0)


def _tc_call(flat, row_start):
    n_rows = flat.shape[0]
    nblk = (n_rows - row_start) // _ROWS_PER_BLOCK
    blk0 = row_start // _ROWS_PER_BLOCK
    return pl.pallas_call(
        _tc_topk_kernel,
        grid=(nblk,),
        in_specs=[pl.BlockSpec((_ROWS_PER_BLOCK, _NCOLS),
                               lambda i: (i + blk0, 0))],
        out_specs=pl.BlockSpec((_ROWS_PER_BLOCK, _NCOLS),
                               lambda i: (i + blk0, 0)),
        out_shape=jax.ShapeDtypeStruct((n_rows, _NCOLS), flat.dtype),
    )(flat)


_ABS_MASK = 0x7FFFFFFF


def _sc_scan_level(hist_ref, nb, need):
    nchunks = nb // 16
    cum = jnp.int32(0)
    done = jnp.bool_(False)
    chosen_c = jnp.int32(0)
    base_above = jnp.int32(0)
    for c in range(nchunks - 1, -1, -1):
        sv = hist_ref[pl.ds(c * 16, 16)]
        s = jnp.sum(sv)
        hit = jnp.logical_and(jnp.logical_not(done), cum + s >= need)
        chosen_c = jnp.where(hit, jnp.int32(c), chosen_c)
        base_above = jnp.where(hit, cum, base_above)
        done = jnp.logical_or(done, hit)
        cum = cum + s
    need_local = need - base_above
    sv = hist_ref[pl.ds(chosen_c * 16, 16)]
    rchunk = lax.rev(sv, dimensions=(0,))
    cs = plsc.cumsum(rchunk)
    first = jnp.logical_and(cs >= need_local, cs - rchunk < need_local)
    zeros = jnp.zeros((16,), jnp.int32)
    in_bucket = jnp.sum(jnp.where(first, rchunk, zeros))
    above_in_chunk = jnp.sum(jnp.where(first, cs - rchunk, zeros))
    jrev = jnp.sum(jnp.where(first, lax.iota(jnp.int32, 16), zeros))
    bucket = chosen_c * 16 + (jnp.int32(15) - jrev)
    return bucket, base_above + above_in_chunk, in_bucket


_CHUNK = 8


def _sc_body(x_hbm, o_hbm, row_v, out_v, hist_ref):
    nc = 2
    wid = lax.axis_index("s") * nc + lax.axis_index("c")
    rows_per_w = _SC_ROWS // _N_WORKERS
    ones16 = jnp.ones((16,), jnp.int32)

    def do_row(rr, carry):
        levels = ((23, 256, None), (15, 256, 23), (7, 256, 15), (0, 128, 7))
        pfx = jnp.int32(0)
        need = jnp.int32(_K)
        above_total = jnp.int32(0)
        in_bucket = jnp.int32(0)
        for shift, nb, cmp_shift in levels:
            for j in range(nb // 16):
                hist_ref[pl.ds(j * 16, 16)] = jnp.zeros((16,), jnp.int32)

            def hist_step(i, c, shift=shift, nb=nb, cmp_shift=cmp_shift,
                          pfx=pfx):
                v = row_v[rr, pl.ds(i * 16, 16)]
                k = jax.lax.bitcast_convert_type(v, jnp.int32) & jnp.int32(
                    _ABS_MASK)
                d = (k >> shift) & jnp.int32(nb - 1)
                if cmp_shift is None:
                    m = k >= jnp.int32(0)
                else:
                    m = (k >> cmp_shift) == pfx
                plsc.addupdate_scatter(hist_ref, [d], ones16, mask=m)
                return c

            lax.fori_loop(0, _NCOLS // 16, hist_step, jnp.int32(0), unroll=8)
            bucket, above, inb = _sc_scan_level(hist_ref, nb, need)
            pfx = (pfx << (8 if nb == 256 else 7)) | bucket
            need = need - above
            above_total = above_total + above
            in_bucket = inb

        thresh = pfx
        total = above_total + in_bucket
        total_v = jnp.zeros((16,), jnp.int32) + total
        scale = jnp.full((16,), float(_NCOLS), jnp.float32) / total_v.astype(
            jnp.float32)

        def out_step(i, c):
            v = row_v[rr, pl.ds(i * 16, 16)]
            k = jax.lax.bitcast_convert_type(v, jnp.int32) & jnp.int32(
                _ABS_MASK)
            out_v[rr, pl.ds(i * 16, 16)] = jnp.where(
                k >= thresh, v * scale, 0.0)
            return c

        lax.fori_loop(0, _NCOLS // 16, out_step, jnp.int32(0), unroll=8)
        return carry

    def do_chunk(g, carry):
        base = wid * rows_per_w + g * _CHUNK
        pltpu.sync_copy(x_hbm.at[pl.ds(base, _CHUNK)], row_v)
        lax.fori_loop(0, _CHUNK, do_row, jnp.int32(0))
        pltpu.sync_copy(out_v, o_hbm.at[pl.ds(base, _CHUNK)])
        return carry

    lax.fori_loop(0, rows_per_w // _CHUNK, do_chunk, jnp.int32(0))


def _sc_call(flat):
    mesh = plsc.VectorSubcoreMesh(core_axis_name="c", subcore_axis_name="s")
    f = pl.kernel(
        _sc_body,
        out_type=jax.ShapeDtypeStruct((_SC_ROWS, _NCOLS), jnp.float32),
        mesh=mesh,
        scratch_types=[
            pltpu.VMEM((_CHUNK, _NCOLS), jnp.float32),
            pltpu.VMEM((_CHUNK, _NCOLS), jnp.float32),
            pltpu.VMEM((256,), jnp.int32),
        ],
        compiler_params=pltpu.CompilerParams(needs_layout_passes=False),
    )
    return f(flat)


def kernel(x):
    shape = x.shape
    flat = x.reshape(-1, shape[-1])
    n_rows = flat.shape[0]
    if _SC_ROWS >= n_rows:
        out = _sc_call(flat)
    elif _SC_ROWS == 0:
        out = _tc_call(flat, 0)
    else:
        sc_part = _sc_call(flat)
        out = _tc_call(flat, _SC_ROWS)
        out = lax.dynamic_update_slice(out, sc_part, (0, 0))
    return out.reshape(shape), 0, 0
